# baseline (speedup 1.0000x reference)
import functools

import jax
import jax.numpy as jnp
from jax import lax
from jax.experimental import pallas as pl
from jax.experimental.pallas import tpu as pltpu
from jax.experimental.pallas import tpu_sc as plsc

N = 10000
E = 320000
D = 128
NRBF = 20
CUTOFF = 12.0
NSP = 64

NC = 2
NS = 16
L = 16
NW = NC * NS
EPT = E // NS
CH1 = 2000
B2 = 64
SURV_CAP = EPT + 128
N_PAD = 10240
N_HALF = N_PAD // NC
ROWS_PT = N_HALF // NS

_SIGMA = CUTOFF / NRBF
_NEG_INV_2S2 = -1.0 / (2.0 * _SIGMA * _SIGMA)
_DELTA = CUTOFF / (NRBF - 1)
_PI = 3.14159265358979323846
_CUT2 = CUTOFF * CUTOFF
_DPACK = 8192
_PADV = 1 << 30
G = 128
_GSCALE = (G - 1) / CUTOFF

_RB = 1000


def _d1_body(an_ref, emb_ref, w1_ref, b1_ref, w2_ref, b2_ref, q_ref, phi_ref):
    an = an_ref[...]
    io = lax.broadcasted_iota(jnp.int32, (_RB, 128), 1)
    oh = (io == an).astype(jnp.float32)
    q = jnp.dot(oh, emb_ref[...], preferred_element_type=jnp.float32)
    h = jnp.dot(q, w1_ref[...], preferred_element_type=jnp.float32) + b1_ref[...]
    h = h * jax.nn.sigmoid(h)
    phi_ref[...] = (jnp.dot(h, w2_ref[...], preferred_element_type=jnp.float32)
                    + b2_ref[...])
    q_ref[...] = q


def _dense1(an2, emb_p, W1, b1r, W2a, b2r):
    return pl.pallas_call(
        _d1_body,
        grid=(N // _RB,),
        in_specs=[
            pl.BlockSpec((_RB, 1), lambda i: (i, 0)),
            pl.BlockSpec((128, D), lambda i: (0, 0)),
            pl.BlockSpec((D, D), lambda i: (0, 0)),
            pl.BlockSpec((1, D), lambda i: (0, 0)),
            pl.BlockSpec((D, D), lambda i: (0, 0)),
            pl.BlockSpec((1, D), lambda i: (0, 0)),
        ],
        out_specs=[
            pl.BlockSpec((_RB, D), lambda i: (i, 0)),
            pl.BlockSpec((_RB, D), lambda i: (i, 0)),
        ],
        out_shape=[
            jax.ShapeDtypeStruct((N, D), jnp.float32),
            jax.ShapeDtypeStruct((N, D), jnp.float32),
        ],
    )(an2, emb_p, W1, b1r, W2a, b2r)


def _d2_body(q_ref, parts_ref, wproj_ref, af_ref, qn_ref):
    qn = q_ref[...] + parts_ref[...]
    af_ref[...] = jnp.dot(qn, wproj_ref[...], preferred_element_type=jnp.float32)
    qn_ref[...] = qn


def _dense2(q, parts, Wproj):
    return pl.pallas_call(
        _d2_body,
        grid=(N // _RB,),
        in_specs=[
            pl.BlockSpec((_RB, D), lambda i: (i, 0)),
            pl.BlockSpec((_RB, D), lambda i: (i, 0)),
            pl.BlockSpec((D, D), lambda i: (0, 0)),
        ],
        out_specs=[
            pl.BlockSpec((_RB, D), lambda i: (i, 0)),
            pl.BlockSpec((_RB, D), lambda i: (i, 0)),
        ],
        out_shape=[
            jax.ShapeDtypeStruct((N, D), jnp.float32),
            jax.ShapeDtypeStruct((N, D), jnp.float32),
        ],
    )(q, parts, Wproj)


def _tab_body(wfa_ref, tab_ref):
    dg = (lax.broadcasted_iota(jnp.int32, (G, 32), 0).astype(jnp.float32)
          * (CUTOFF / (G - 1)))
    ci = lax.broadcasted_iota(jnp.int32, (G, 32), 1)
    cif = ci.astype(jnp.float32)
    rbf = jnp.exp(-((dg - cif * _DELTA) ** 2) * (-_NEG_INV_2S2))
    basis = jnp.where(ci < NRBF, rbf,
                      jnp.where(ci == NRBF, 1.0, 0.0))
    dgc = dg[:, :1]
    fc = 0.5 * (jnp.cos(dgc * (_PI / CUTOFF)) + 1.0)
    fc = fc * (dgc < CUTOFF).astype(jnp.float32)
    t = jnp.dot(basis, wfa_ref[...], preferred_element_type=jnp.float32) * fc
    rows = lax.broadcasted_iota(jnp.int32, (G, D), 0)
    tnext = jnp.concatenate([lax.slice(t, (1, 0), (G, D)),
                             jnp.zeros((1, D), jnp.float32)], axis=0)
    delta = (tnext - t) * (rows < G - 1).astype(jnp.float32)
    hi = lax.bitcast_convert_type(t.astype(jnp.bfloat16),
                                  jnp.uint16).astype(jnp.int32)
    lo = lax.bitcast_convert_type(delta.astype(jnp.bfloat16),
                                  jnp.uint16).astype(jnp.int32)
    tab_ref[...] = (hi << 16) | lo


def _build_table(wfa_pad):
    return pl.pallas_call(
        _tab_body,
        grid=(1,),
        in_specs=[pl.BlockSpec((32, D), lambda i: (0, 0))],
        out_specs=pl.BlockSpec((G, D), lambda i: (0, 0)),
        out_shape=jax.ShapeDtypeStruct((G, D), jnp.int32),
    )(wfa_pad)


def _edge_body(src_hbm, dst_hbm, posx_hbm, posy_hbm, posz_hbm, phi_hbm,
               tab_hbm, zer_hbm, out_hbm,
               posx_v, posy_v, posz_v, tab_v, srcbuf, dstbuf,
               spack, srcidx, idx2d, phibuf, outbuf, sem, acc):
    cid = lax.axis_index("c")
    sid = lax.axis_index("s")
    lo = cid * N_HALF

    pltpu.sync_copy(posx_hbm, posx_v)
    pltpu.sync_copy(posy_hbm, posy_v)
    pltpu.sync_copy(posz_hbm, posz_v)
    pltpu.sync_copy(tab_hbm, tab_v)
    coffs = [lax.iota(jnp.int32, L) + c * L for c in range(D // L)]
    pltpu.sync_copy(zer_hbm, acc.at[pl.ds(sid * ROWS_PT, ROWS_PT)])

    ebase = sid * EPT

    def chunk1(c, cnt):
        pltpu.sync_copy(src_hbm.at[pl.ds(ebase + c * CH1, CH1)], srcbuf)
        pltpu.sync_copy(dst_hbm.at[pl.ds(ebase + c * CH1, CH1)], dstbuf)

        def vreg5(i5, cnt):
            for u in range(5):
                cnt = scan16(i5 * 5 + u, cnt)
            return cnt

        def scan16(i, cnt):
            s16 = srcbuf[pl.ds(i * L, L)]
            d16 = dstbuf[pl.ds(i * L, L)]
            xs = plsc.load_gather(posx_v, [s16])
            xd = plsc.load_gather(posx_v, [d16])
            ys = plsc.load_gather(posy_v, [s16])
            yd = plsc.load_gather(posy_v, [d16])
            zs = plsc.load_gather(posz_v, [s16])
            zd = plsc.load_gather(posz_v, [d16])
            dx = xd - xs
            dy = yd - ys
            dz = zd - zs
            d2 = dx * dx + dy * dy + dz * dz + 1e-12
            d16r = d16 - lo
            m = ((d2 < _CUT2) & (d16r >= 0)) & (d16r < N_HALF)
            pk = (s16 << 13) | d16r
            plsc.store_compressed(spack.at[pl.ds(cnt, L)], pk, mask=m)
            cntv = plsc.all_reduce_population_count(m)
            return cnt + cntv[0]

        return lax.fori_loop(0, CH1 // L // 5, vreg5, cnt)

    cnt = lax.fori_loop(0, EPT // CH1, chunk1, jnp.int32(0))

    padv = jnp.full((L,), _PADV, jnp.int32)
    for j in range(B2 // L):
        spack[pl.ds(cnt + j * L, L)] = padv

    plsc.subcore_barrier()

    nch = (cnt + B2 - 1) // B2

    def chunk2(ch, _):
        base = ch * B2
        for j in range(B2 // L):
            pk = spack[pl.ds(base + j * L, L)]
            m0 = pk < _PADV
            srcidx[pl.ds(j * L, L)] = jnp.where(m0, pk >> 13, 0)
            idx2d[0, pl.ds(j * L, L)] = jnp.where(m0, pk & (_DPACK - 1), 0)
        pltpu.async_copy(phi_hbm.at[srcidx], phibuf, sem).wait()

        @plsc.parallel_loop(0, B2 // L, unroll=2)
        def group(g):
            pk = spack[pl.ds(base + g * L, L)]
            m = pk < _PADV
            s16 = srcidx[pl.ds(g * L, L)]
            d16 = idx2d[0, pl.ds(g * L, L)] + lo
            xs = plsc.load_gather(posx_v, [s16])
            xd = plsc.load_gather(posx_v, [d16])
            ys = plsc.load_gather(posy_v, [s16])
            yd = plsc.load_gather(posy_v, [d16])
            zs = plsc.load_gather(posz_v, [s16])
            zd = plsc.load_gather(posz_v, [d16])
            dx = xd - xs
            dy = yd - ys
            dz = zd - zs
            d2v = dx * dx + dy * dy + dz * dz + 1e-12
            ii = plsc.bitcast(d2v, jnp.int32)
            y = plsc.bitcast(jnp.int32(0x5F3759DF) - (ii >> 1), jnp.float32)
            for _i in range(3):
                y = y * (1.5 - 0.5 * d2v * y * y)
            dist = d2v * y
            u = dist * _GSCALE
            iv = u.astype(jnp.int32)
            isel = jnp.where(m, iv, G - 1)
            frac = u - isel.astype(jnp.float32)

            for j in range(L):
                row = g * L + j
                sel = jnp.zeros((L,), jnp.int32) + j
                ib = isel.at[sel].get(mode="promise_in_bounds")
                fb = frac.at[sel].get(mode="promise_in_bounds")
                for c in range(D // L):
                    w16 = plsc.load_gather(tab_v, [ib, coffs[c]])
                    val = plsc.bitcast(w16 & jnp.int32(-65536), jnp.float32)
                    dlt = plsc.bitcast(w16 << 16, jnp.float32)
                    wij = val + fb * dlt
                    outbuf[row, pl.ds(c * L, L)] = (
                        phibuf[row, pl.ds(c * L, L)] * wij)

        pltpu.sync_copy(outbuf, acc.at[idx2d.at[0]], add=True)
        return 0

    lax.fori_loop(0, nch, chunk2, 0)

    plsc.subcore_barrier()
    pltpu.sync_copy(acc.at[pl.ds(sid * ROWS_PT, ROWS_PT)],
                    out_hbm.at[pl.ds(lo + sid * ROWS_PT, ROWS_PT)])


def _sc_edges(src, dst, posx, posy, posz, phi, tab, zer):
    mesh = plsc.VectorSubcoreMesh(core_axis_name="c", subcore_axis_name="s")
    f = functools.partial(
        pl.kernel, _edge_body,
        out_type=jax.ShapeDtypeStruct((N_PAD, D), jnp.float32),
        mesh=mesh,
        compiler_params=pltpu.CompilerParams(needs_layout_passes=False),
        scratch_types=[
            pltpu.VMEM((N,), jnp.float32),
            pltpu.VMEM((N,), jnp.float32),
            pltpu.VMEM((N,), jnp.float32),
            pltpu.VMEM((G, D), jnp.int32),
            pltpu.VMEM((CH1,), jnp.int32),
            pltpu.VMEM((CH1,), jnp.int32),
            pltpu.VMEM((SURV_CAP,), jnp.int32),
            pltpu.VMEM((B2,), jnp.int32),
            pltpu.VMEM((1, B2), jnp.int32),
            pltpu.VMEM((B2, D), jnp.float32),
            pltpu.VMEM((B2, D), jnp.float32),
            pltpu.SemaphoreType.DMA,
            pltpu.VMEM_SHARED((N_HALF, D), jnp.float32),
        ],
    )()
    return f(src, dst, posx, posy, posz, phi, tab, zer)


def _spg_body(qn_hbm, idx_hbm, out_hbm, idx_v, rows_v, sem):
    cid = lax.axis_index("c")
    sid = lax.axis_index("s")
    wid = sid * NC + cid

    @pl.when(wid < NSP // 8)
    def _():
        base = wid * 8
        pltpu.sync_copy(idx_hbm.at[pl.ds(base, 8)], idx_v)
        pltpu.async_copy(qn_hbm.at[idx_v], rows_v, sem).wait()
        pltpu.sync_copy(rows_v, out_hbm.at[pl.ds(base, 8)])


def _sc_spgather(qn, spidx):
    mesh = plsc.VectorSubcoreMesh(core_axis_name="c", subcore_axis_name="s")
    f = functools.partial(
        pl.kernel, _spg_body,
        out_type=jax.ShapeDtypeStruct((NSP, D), jnp.float32),
        mesh=mesh,
        compiler_params=pltpu.CompilerParams(needs_layout_passes=False),
        scratch_types=[
            pltpu.VMEM((8,), jnp.int32),
            pltpu.VMEM((8, D), jnp.float32),
            pltpu.SemaphoreType.DMA,
        ],
    )()
    return f(qn, spidx)


def kernel(pos, atomic_numbers, edge_index, spnode_idx, emb, W1, b1, W2, b2,
           Wf, bf, Wproj):
    f32 = jnp.float32
    src = edge_index[0].astype(jnp.int32)
    dst = edge_index[1].astype(jnp.int32)
    posx = pos[:, 0].astype(f32)
    posy = pos[:, 1].astype(f32)
    posz = pos[:, 2].astype(f32)
    emb_p = jnp.zeros((128, D), f32).at[:emb.shape[0]].set(emb)
    W2a = W2[:, :D]
    b2a = b2[:D].reshape(1, D)
    wfa_pad = jnp.concatenate(
        [Wf[:, :D], bf[None, :D], jnp.zeros((32 - NRBF - 1, D), f32)], axis=0)
    zer = jnp.zeros((ROWS_PT, D), f32)
    an2 = atomic_numbers.astype(jnp.int32).reshape(N, 1)

    q, phi = _dense1(an2, emb_p, W1, b1.reshape(1, D), W2a, b2a)
    tab = _build_table(wfa_pad)
    parts = _sc_edges(src, dst, posx, posy, posz, phi, tab, zer)
    atom_feat, qn = _dense2(q, parts, Wproj)
    spnode_feat = _sc_spgather(qn, spnode_idx.astype(jnp.int32))
    return atom_feat, spnode_feat


# speedup vs baseline: 1.0371x; 1.0371x over previous
import functools

import jax
import jax.numpy as jnp
from jax import lax
from jax.experimental import pallas as pl
from jax.experimental.pallas import tpu as pltpu
from jax.experimental.pallas import tpu_sc as plsc

N = 10000
E = 320000
D = 128
NRBF = 20
CUTOFF = 12.0
NSP = 64

NC = 2
NS = 16
L = 16
NW = NC * NS
EPT = E // NS
CH1 = 2000
B2 = 64
SURV_CAP = EPT + 128
N_PAD = 10240
N_HALF = N_PAD // NC
ROWS_PT = N_HALF // NS

_SIGMA = CUTOFF / NRBF
_NEG_INV_2S2 = -1.0 / (2.0 * _SIGMA * _SIGMA)
_DELTA = CUTOFF / (NRBF - 1)
_PI = 3.14159265358979323846
_CUT2 = CUTOFF * CUTOFF
_DPACK = 8192
_PADV = 1 << 30
G = 128
_GSCALE = (G - 1) / CUTOFF

_RB = 1000


def _d1_body(an_ref, emb_ref, w1_ref, b1_ref, w2_ref, b2_ref, q_ref, phi_ref):
    an = an_ref[...]
    io = lax.broadcasted_iota(jnp.int32, (_RB, 128), 1)
    oh = (io == an).astype(jnp.float32)
    q = jnp.dot(oh, emb_ref[...], preferred_element_type=jnp.float32)
    h = jnp.dot(q, w1_ref[...], preferred_element_type=jnp.float32) + b1_ref[...]
    h = h * jax.nn.sigmoid(h)
    phi_ref[...] = (jnp.dot(h, w2_ref[...], preferred_element_type=jnp.float32)
                    + b2_ref[...])
    q_ref[...] = q


def _dense1(an2, emb_p, W1, b1r, W2a, b2r):
    return pl.pallas_call(
        _d1_body,
        grid=(N // _RB,),
        in_specs=[
            pl.BlockSpec((_RB, 1), lambda i: (i, 0)),
            pl.BlockSpec((128, D), lambda i: (0, 0)),
            pl.BlockSpec((D, D), lambda i: (0, 0)),
            pl.BlockSpec((1, D), lambda i: (0, 0)),
            pl.BlockSpec((D, D), lambda i: (0, 0)),
            pl.BlockSpec((1, D), lambda i: (0, 0)),
        ],
        out_specs=[
            pl.BlockSpec((_RB, D), lambda i: (i, 0)),
            pl.BlockSpec((_RB, D), lambda i: (i, 0)),
        ],
        out_shape=[
            jax.ShapeDtypeStruct((N, D), jnp.float32),
            jax.ShapeDtypeStruct((N, D), jnp.float32),
        ],
    )(an2, emb_p, W1, b1r, W2a, b2r)


def _d2_body(q_ref, parts_ref, wproj_ref, af_ref, qn_ref):
    qn = q_ref[...] + parts_ref[...]
    af_ref[...] = jnp.dot(qn, wproj_ref[...], preferred_element_type=jnp.float32)
    qn_ref[...] = qn


def _dense2(q, parts, Wproj):
    return pl.pallas_call(
        _d2_body,
        grid=(N // _RB,),
        in_specs=[
            pl.BlockSpec((_RB, D), lambda i: (i, 0)),
            pl.BlockSpec((_RB, D), lambda i: (i, 0)),
            pl.BlockSpec((D, D), lambda i: (0, 0)),
        ],
        out_specs=[
            pl.BlockSpec((_RB, D), lambda i: (i, 0)),
            pl.BlockSpec((_RB, D), lambda i: (i, 0)),
        ],
        out_shape=[
            jax.ShapeDtypeStruct((N, D), jnp.float32),
            jax.ShapeDtypeStruct((N, D), jnp.float32),
        ],
    )(q, parts, Wproj)


def _tab_body(wfa_ref, tab_ref):
    dg = (lax.broadcasted_iota(jnp.int32, (G, 32), 0).astype(jnp.float32)
          * (CUTOFF / (G - 1)))
    ci = lax.broadcasted_iota(jnp.int32, (G, 32), 1)
    cif = ci.astype(jnp.float32)
    rbf = jnp.exp(-((dg - cif * _DELTA) ** 2) * (-_NEG_INV_2S2))
    basis = jnp.where(ci < NRBF, rbf,
                      jnp.where(ci == NRBF, 1.0, 0.0))
    dgc = dg[:, :1]
    fc = 0.5 * (jnp.cos(dgc * (_PI / CUTOFF)) + 1.0)
    fc = fc * (dgc < CUTOFF).astype(jnp.float32)
    t = jnp.dot(basis, wfa_ref[...], preferred_element_type=jnp.float32) * fc
    rows = lax.broadcasted_iota(jnp.int32, (G, D), 0)
    tnext = jnp.concatenate([lax.slice(t, (1, 0), (G, D)),
                             jnp.zeros((1, D), jnp.float32)], axis=0)
    delta = (tnext - t) * (rows < G - 1).astype(jnp.float32)
    hi = lax.bitcast_convert_type(t.astype(jnp.bfloat16),
                                  jnp.uint16).astype(jnp.int32)
    lo = lax.bitcast_convert_type(delta.astype(jnp.bfloat16),
                                  jnp.uint16).astype(jnp.int32)
    tab_ref[...] = (hi << 16) | lo


def _build_table(wfa_pad):
    return pl.pallas_call(
        _tab_body,
        grid=(1,),
        in_specs=[pl.BlockSpec((32, D), lambda i: (0, 0))],
        out_specs=pl.BlockSpec((G, D), lambda i: (0, 0)),
        out_shape=jax.ShapeDtypeStruct((G, D), jnp.int32),
    )(wfa_pad)


def _edge_body(src_hbm, dst_hbm, posx_hbm, posy_hbm, posz_hbm, phi_hbm,
               tab_hbm, zer_hbm, out_hbm,
               posx_v, posy_v, posz_v, tab_v, srcbuf, dstbuf,
               spack, srcidx, idx2d, phibuf, outbuf, sem, acc):
    cid = lax.axis_index("c")
    sid = lax.axis_index("s")
    lo = cid * N_HALF

    pltpu.sync_copy(posx_hbm, posx_v)
    pltpu.sync_copy(posy_hbm, posy_v)
    pltpu.sync_copy(posz_hbm, posz_v)
    pltpu.sync_copy(tab_hbm, tab_v)
    coffs = [lax.iota(jnp.int32, L) + c * L for c in range(D // L)]
    pltpu.sync_copy(zer_hbm, acc.at[pl.ds(sid * ROWS_PT, ROWS_PT)])

    ebase = sid * EPT

    def chunk1(c, cnt):
        pltpu.sync_copy(src_hbm.at[pl.ds(ebase + c * CH1, CH1)], srcbuf)
        pltpu.sync_copy(dst_hbm.at[pl.ds(ebase + c * CH1, CH1)], dstbuf)

        def vreg5(i5, cnt):
            for u in range(5):
                cnt = scan16(i5 * 5 + u, cnt)
            return cnt

        def scan16(i, cnt):
            s16 = srcbuf[pl.ds(i * L, L)]
            d16 = dstbuf[pl.ds(i * L, L)]
            xs = plsc.load_gather(posx_v, [s16])
            xd = plsc.load_gather(posx_v, [d16])
            ys = plsc.load_gather(posy_v, [s16])
            yd = plsc.load_gather(posy_v, [d16])
            zs = plsc.load_gather(posz_v, [s16])
            zd = plsc.load_gather(posz_v, [d16])
            dx = xd - xs
            dy = yd - ys
            dz = zd - zs
            d2 = dx * dx + dy * dy + dz * dz + 1e-12
            d16r = d16 - lo
            m = ((d2 < _CUT2) & (d16r >= 0)) & (d16r < N_HALF)
            pk = (s16 << 13) | d16r
            plsc.store_compressed(spack.at[pl.ds(cnt, L)], pk, mask=m)
            cntv = plsc.all_reduce_population_count(m)
            return cnt + cntv[0]

        return lax.fori_loop(0, CH1 // L // 5, vreg5, cnt)

    cnt = lax.fori_loop(0, EPT // CH1, chunk1, jnp.int32(0))

    padv = jnp.full((L,), _PADV, jnp.int32)
    for j in range(B2 // L):
        spack[pl.ds(cnt + j * L, L)] = padv

    plsc.subcore_barrier()

    nch = (cnt + B2 - 1) // B2

    def chunk2(ch, _):
        base = ch * B2
        for j in range(B2 // L):
            pk = spack[pl.ds(base + j * L, L)]
            m0 = pk < _PADV
            srcidx[pl.ds(j * L, L)] = jnp.where(m0, pk >> 13, 0)
            idx2d[0, pl.ds(j * L, L)] = jnp.where(m0, pk & (_DPACK - 1), 0)
        pltpu.async_copy(phi_hbm.at[srcidx], phibuf, sem).wait()

        @plsc.parallel_loop(0, B2 // L, unroll=4)
        def group(g):
            pk = spack[pl.ds(base + g * L, L)]
            m = pk < _PADV
            s16 = srcidx[pl.ds(g * L, L)]
            d16 = idx2d[0, pl.ds(g * L, L)] + lo
            xs = plsc.load_gather(posx_v, [s16])
            xd = plsc.load_gather(posx_v, [d16])
            ys = plsc.load_gather(posy_v, [s16])
            yd = plsc.load_gather(posy_v, [d16])
            zs = plsc.load_gather(posz_v, [s16])
            zd = plsc.load_gather(posz_v, [d16])
            dx = xd - xs
            dy = yd - ys
            dz = zd - zs
            d2v = dx * dx + dy * dy + dz * dz + 1e-12
            ii = plsc.bitcast(d2v, jnp.int32)
            y = plsc.bitcast(jnp.int32(0x5F3759DF) - (ii >> 1), jnp.float32)
            for _i in range(3):
                y = y * (1.5 - 0.5 * d2v * y * y)
            dist = d2v * y
            u = dist * _GSCALE
            iv = u.astype(jnp.int32)
            isel = jnp.where(m, iv, G - 1)
            frac = u - isel.astype(jnp.float32)

            for j in range(L):
                row = g * L + j
                sel = jnp.zeros((L,), jnp.int32) + j
                ib = isel.at[sel].get(mode="promise_in_bounds")
                fb = frac.at[sel].get(mode="promise_in_bounds")
                for c in range(D // L):
                    w16 = plsc.load_gather(tab_v, [ib, coffs[c]])
                    val = plsc.bitcast(w16 & jnp.int32(-65536), jnp.float32)
                    dlt = plsc.bitcast(w16 << 16, jnp.float32)
                    wij = val + fb * dlt
                    outbuf[row, pl.ds(c * L, L)] = (
                        phibuf[row, pl.ds(c * L, L)] * wij)

        pltpu.sync_copy(outbuf, acc.at[idx2d.at[0]], add=True)
        return 0

    lax.fori_loop(0, nch, chunk2, 0)

    plsc.subcore_barrier()
    pltpu.sync_copy(acc.at[pl.ds(sid * ROWS_PT, ROWS_PT)],
                    out_hbm.at[pl.ds(lo + sid * ROWS_PT, ROWS_PT)])


def _sc_edges(src, dst, posx, posy, posz, phi, tab, zer):
    mesh = plsc.VectorSubcoreMesh(core_axis_name="c", subcore_axis_name="s")
    f = functools.partial(
        pl.kernel, _edge_body,
        out_type=jax.ShapeDtypeStruct((N_PAD, D), jnp.float32),
        mesh=mesh,
        compiler_params=pltpu.CompilerParams(needs_layout_passes=False),
        scratch_types=[
            pltpu.VMEM((N,), jnp.float32),
            pltpu.VMEM((N,), jnp.float32),
            pltpu.VMEM((N,), jnp.float32),
            pltpu.VMEM((G, D), jnp.int32),
            pltpu.VMEM((CH1,), jnp.int32),
            pltpu.VMEM((CH1,), jnp.int32),
            pltpu.VMEM((SURV_CAP,), jnp.int32),
            pltpu.VMEM((B2,), jnp.int32),
            pltpu.VMEM((1, B2), jnp.int32),
            pltpu.VMEM((B2, D), jnp.float32),
            pltpu.VMEM((B2, D), jnp.float32),
            pltpu.SemaphoreType.DMA,
            pltpu.VMEM_SHARED((N_HALF, D), jnp.float32),
        ],
    )()
    return f(src, dst, posx, posy, posz, phi, tab, zer)


def _spg_body(qn_hbm, idx_hbm, out_hbm, idx_v, rows_v, sem):
    cid = lax.axis_index("c")
    sid = lax.axis_index("s")
    wid = sid * NC + cid

    @pl.when(wid < NSP // 8)
    def _():
        base = wid * 8
        pltpu.sync_copy(idx_hbm.at[pl.ds(base, 8)], idx_v)
        pltpu.async_copy(qn_hbm.at[idx_v], rows_v, sem).wait()
        pltpu.sync_copy(rows_v, out_hbm.at[pl.ds(base, 8)])


def _sc_spgather(qn, spidx):
    mesh = plsc.VectorSubcoreMesh(core_axis_name="c", subcore_axis_name="s")
    f = functools.partial(
        pl.kernel, _spg_body,
        out_type=jax.ShapeDtypeStruct((NSP, D), jnp.float32),
        mesh=mesh,
        compiler_params=pltpu.CompilerParams(needs_layout_passes=False),
        scratch_types=[
            pltpu.VMEM((8,), jnp.int32),
            pltpu.VMEM((8, D), jnp.float32),
            pltpu.SemaphoreType.DMA,
        ],
    )()
    return f(qn, spidx)


def kernel(pos, atomic_numbers, edge_index, spnode_idx, emb, W1, b1, W2, b2,
           Wf, bf, Wproj):
    f32 = jnp.float32
    src = edge_index[0].astype(jnp.int32)
    dst = edge_index[1].astype(jnp.int32)
    posx = pos[:, 0].astype(f32)
    posy = pos[:, 1].astype(f32)
    posz = pos[:, 2].astype(f32)
    emb_p = jnp.zeros((128, D), f32).at[:emb.shape[0]].set(emb)
    W2a = W2[:, :D]
    b2a = b2[:D].reshape(1, D)
    wfa_pad = jnp.concatenate(
        [Wf[:, :D], bf[None, :D], jnp.zeros((32 - NRBF - 1, D), f32)], axis=0)
    zer = jnp.zeros((ROWS_PT, D), f32)
    an2 = atomic_numbers.astype(jnp.int32).reshape(N, 1)

    q, phi = _dense1(an2, emb_p, W1, b1.reshape(1, D), W2a, b2a)
    tab = _build_table(wfa_pad)
    parts = _sc_edges(src, dst, posx, posy, posz, phi, tab, zer)
    atom_feat, qn = _dense2(q, parts, Wproj)
    spnode_feat = _sc_spgather(qn, spnode_idx.astype(jnp.int32))
    return atom_feat, spnode_feat


# speedup vs baseline: 1.4579x; 1.4057x over previous
import functools

import jax
import jax.numpy as jnp
from jax import lax
from jax.experimental import pallas as pl
from jax.experimental.pallas import tpu as pltpu
from jax.experimental.pallas import tpu_sc as plsc

N = 10000
E = 320000
D = 128
NRBF = 20
CUTOFF = 12.0
NSP = 64

NC = 2
NS = 16
L = 16
NW = NC * NS
EPT = E // NS
CH1 = 2000
B2 = 64
SURV_CAP = EPT + 128
N_PAD = 10240
N_HALF = N_PAD // NC
ROWS_PT = N_HALF // NS

_SIGMA = CUTOFF / NRBF
_NEG_INV_2S2 = -1.0 / (2.0 * _SIGMA * _SIGMA)
_DELTA = CUTOFF / (NRBF - 1)
_PI = 3.14159265358979323846
_CUT2 = CUTOFF * CUTOFF
_DPACK = 8192
_PADV = 1 << 30
G = 128
_GSCALE = (G - 1) / CUTOFF

_RB = 1000


def _d1_body(an_ref, emb_ref, w1_ref, b1_ref, w2_ref, b2_ref, q_ref, phi_ref):
    an = an_ref[...]
    io = lax.broadcasted_iota(jnp.int32, (_RB, 128), 1)
    oh = (io == an).astype(jnp.float32)
    q = jnp.dot(oh, emb_ref[...], preferred_element_type=jnp.float32)
    h = jnp.dot(q, w1_ref[...], preferred_element_type=jnp.float32) + b1_ref[...]
    h = h * jax.nn.sigmoid(h)
    phi_ref[...] = (jnp.dot(h, w2_ref[...], preferred_element_type=jnp.float32)
                    + b2_ref[...])
    q_ref[...] = q


def _dense1(an2, emb_p, W1, b1r, W2a, b2r):
    return pl.pallas_call(
        _d1_body,
        grid=(N // _RB,),
        in_specs=[
            pl.BlockSpec((_RB, 1), lambda i: (i, 0)),
            pl.BlockSpec((128, D), lambda i: (0, 0)),
            pl.BlockSpec((D, D), lambda i: (0, 0)),
            pl.BlockSpec((1, D), lambda i: (0, 0)),
            pl.BlockSpec((D, D), lambda i: (0, 0)),
            pl.BlockSpec((1, D), lambda i: (0, 0)),
        ],
        out_specs=[
            pl.BlockSpec((_RB, D), lambda i: (i, 0)),
            pl.BlockSpec((_RB, D), lambda i: (i, 0)),
        ],
        out_shape=[
            jax.ShapeDtypeStruct((N, D), jnp.float32),
            jax.ShapeDtypeStruct((N, D), jnp.float32),
        ],
    )(an2, emb_p, W1, b1r, W2a, b2r)


def _d2_body(q_ref, parts_ref, wproj_ref, af_ref, qn_ref):
    qn = q_ref[...] + parts_ref[...]
    af_ref[...] = jnp.dot(qn, wproj_ref[...], preferred_element_type=jnp.float32)
    qn_ref[...] = qn


def _dense2(q, parts, Wproj):
    return pl.pallas_call(
        _d2_body,
        grid=(N // _RB,),
        in_specs=[
            pl.BlockSpec((_RB, D), lambda i: (i, 0)),
            pl.BlockSpec((_RB, D), lambda i: (i, 0)),
            pl.BlockSpec((D, D), lambda i: (0, 0)),
        ],
        out_specs=[
            pl.BlockSpec((_RB, D), lambda i: (i, 0)),
            pl.BlockSpec((_RB, D), lambda i: (i, 0)),
        ],
        out_shape=[
            jax.ShapeDtypeStruct((N, D), jnp.float32),
            jax.ShapeDtypeStruct((N, D), jnp.float32),
        ],
    )(q, parts, Wproj)


def _tab_body(wfa_ref, tab_ref):
    dg = (lax.broadcasted_iota(jnp.int32, (G, 32), 0).astype(jnp.float32)
          * (CUTOFF / (G - 1)))
    ci = lax.broadcasted_iota(jnp.int32, (G, 32), 1)
    cif = ci.astype(jnp.float32)
    rbf = jnp.exp(-((dg - cif * _DELTA) ** 2) * (-_NEG_INV_2S2))
    basis = jnp.where(ci < NRBF, rbf,
                      jnp.where(ci == NRBF, 1.0, 0.0))
    dgc = dg[:, :1]
    fc = 0.5 * (jnp.cos(dgc * (_PI / CUTOFF)) + 1.0)
    fc = fc * (dgc < CUTOFF).astype(jnp.float32)
    t = jnp.dot(basis, wfa_ref[...], preferred_element_type=jnp.float32) * fc
    rows = lax.broadcasted_iota(jnp.int32, (G, D), 0)
    tnext = jnp.concatenate([lax.slice(t, (1, 0), (G, D)),
                             jnp.zeros((1, D), jnp.float32)], axis=0)
    delta = (tnext - t) * (rows < G - 1).astype(jnp.float32)
    hi = lax.bitcast_convert_type(t.astype(jnp.bfloat16),
                                  jnp.uint16).astype(jnp.int32)
    lo = lax.bitcast_convert_type(delta.astype(jnp.bfloat16),
                                  jnp.uint16).astype(jnp.int32)
    tab_ref[...] = (hi << 16) | lo


def _build_table(wfa_pad):
    return pl.pallas_call(
        _tab_body,
        grid=(1,),
        in_specs=[pl.BlockSpec((32, D), lambda i: (0, 0))],
        out_specs=pl.BlockSpec((G, D), lambda i: (0, 0)),
        out_shape=jax.ShapeDtypeStruct((G, D), jnp.int32),
    )(wfa_pad)


def _edge_body(src_hbm, dst_hbm, posx_hbm, posy_hbm, posz_hbm, phi_hbm,
               tab_hbm, zer_hbm, out_hbm,
               posx_v, posy_v, posz_v, tab_v, srcbuf, dstbuf,
               spack, srcidx, idx2d, phibuf, outbuf, sem, acc):
    cid = lax.axis_index("c")
    sid = lax.axis_index("s")
    lo = cid * N_HALF

    pltpu.sync_copy(posx_hbm, posx_v)
    pltpu.sync_copy(posy_hbm, posy_v)
    pltpu.sync_copy(posz_hbm, posz_v)
    pltpu.sync_copy(tab_hbm, tab_v)
    coffs = [lax.iota(jnp.int32, L) + c * L for c in range(D // L)]
    pltpu.sync_copy(zer_hbm, acc.at[pl.ds(sid * ROWS_PT, ROWS_PT)])

    ebase = sid * EPT

    def chunk1(c, cnt):
        pltpu.sync_copy(src_hbm.at[pl.ds(ebase + c * CH1, CH1)], srcbuf)
        pltpu.sync_copy(dst_hbm.at[pl.ds(ebase + c * CH1, CH1)], dstbuf)

        def vreg5(i5, cnt):
            for u in range(5):
                cnt = scan16(i5 * 5 + u, cnt)
            return cnt

        def scan16(i, cnt):
            s16 = srcbuf[pl.ds(i * L, L)]
            d16 = dstbuf[pl.ds(i * L, L)]
            xs = plsc.load_gather(posx_v, [s16])
            xd = plsc.load_gather(posx_v, [d16])
            ys = plsc.load_gather(posy_v, [s16])
            yd = plsc.load_gather(posy_v, [d16])
            zs = plsc.load_gather(posz_v, [s16])
            zd = plsc.load_gather(posz_v, [d16])
            dx = xd - xs
            dy = yd - ys
            dz = zd - zs
            d2 = dx * dx + dy * dy + dz * dz + 1e-12
            d16r = d16 - lo
            m = ((d2 < _CUT2) & (d16r >= 0)) & (d16r < N_HALF)
            pk = (s16 << 13) | d16r
            plsc.store_compressed(spack.at[pl.ds(cnt, L)], pk, mask=m)
            cntv = plsc.all_reduce_population_count(m)
            return cnt + cntv[0]

        return lax.fori_loop(0, CH1 // L // 5, vreg5, cnt)

    cnt = lax.fori_loop(0, EPT // CH1, chunk1, jnp.int32(0))

    padv = jnp.full((L,), _PADV, jnp.int32)
    for j in range(B2 // L):
        spack[pl.ds(cnt + j * L, L)] = padv

    plsc.subcore_barrier()

    nch = (cnt + B2 - 1) // B2

    def chunk2(ch, _):
        base = ch * B2
        for j in range(B2 // L):
            pk = spack[pl.ds(base + j * L, L)]
            m0 = pk < _PADV
            srcidx[pl.ds(j * L, L)] = jnp.where(m0, pk >> 13, 0)
            idx2d[0, pl.ds(j * L, L)] = jnp.where(m0, pk & (_DPACK - 1), 0)
        pltpu.async_copy(phi_hbm.at[srcidx], phibuf, sem).wait()

        def group(g, _):
            pk = spack[pl.ds(base + g * L, L)]
            m = pk < _PADV
            s16 = srcidx[pl.ds(g * L, L)]
            d16 = idx2d[0, pl.ds(g * L, L)] + lo
            xs = plsc.load_gather(posx_v, [s16])
            xd = plsc.load_gather(posx_v, [d16])
            ys = plsc.load_gather(posy_v, [s16])
            yd = plsc.load_gather(posy_v, [d16])
            zs = plsc.load_gather(posz_v, [s16])
            zd = plsc.load_gather(posz_v, [d16])
            dx = xd - xs
            dy = yd - ys
            dz = zd - zs
            d2v = dx * dx + dy * dy + dz * dz + 1e-12
            ii = plsc.bitcast(d2v, jnp.int32)
            y = plsc.bitcast(jnp.int32(0x5F3759DF) - (ii >> 1), jnp.float32)
            for _i in range(3):
                y = y * (1.5 - 0.5 * d2v * y * y)
            dist = d2v * y
            u = dist * _GSCALE
            iv = u.astype(jnp.int32)
            isel = jnp.where(m, iv, G - 1)
            frac = u - isel.astype(jnp.float32)

            for j in range(L):
                row = g * L + j
                sel = jnp.zeros((L,), jnp.int32) + j
                ib = isel.at[sel].get(mode="promise_in_bounds")
                fb = frac.at[sel].get(mode="promise_in_bounds")
                ws = [plsc.load_gather(tab_v, [ib, coffs[c]])
                      for c in range(D // L)]
                phs = [phibuf[row, pl.ds(c * L, L)] for c in range(D // L)]
                for c in range(D // L):
                    val = plsc.bitcast(ws[c] & jnp.int32(-65536), jnp.float32)
                    dlt = plsc.bitcast(ws[c] << 16, jnp.float32)
                    outbuf[row, pl.ds(c * L, L)] = phs[c] * (val + fb * dlt)
            return 0

        lax.fori_loop(0, B2 // L, group, 0)
        pltpu.sync_copy(outbuf, acc.at[idx2d.at[0]], add=True)
        return 0

    lax.fori_loop(0, nch, chunk2, 0)

    plsc.subcore_barrier()
    pltpu.sync_copy(acc.at[pl.ds(sid * ROWS_PT, ROWS_PT)],
                    out_hbm.at[pl.ds(lo + sid * ROWS_PT, ROWS_PT)])


def _sc_edges(src, dst, posx, posy, posz, phi, tab, zer):
    mesh = plsc.VectorSubcoreMesh(core_axis_name="c", subcore_axis_name="s")
    f = functools.partial(
        pl.kernel, _edge_body,
        out_type=jax.ShapeDtypeStruct((N_PAD, D), jnp.float32),
        mesh=mesh,
        compiler_params=pltpu.CompilerParams(needs_layout_passes=False),
        scratch_types=[
            pltpu.VMEM((N,), jnp.float32),
            pltpu.VMEM((N,), jnp.float32),
            pltpu.VMEM((N,), jnp.float32),
            pltpu.VMEM((G, D), jnp.int32),
            pltpu.VMEM((CH1,), jnp.int32),
            pltpu.VMEM((CH1,), jnp.int32),
            pltpu.VMEM((SURV_CAP,), jnp.int32),
            pltpu.VMEM((B2,), jnp.int32),
            pltpu.VMEM((1, B2), jnp.int32),
            pltpu.VMEM((B2, D), jnp.float32),
            pltpu.VMEM((B2, D), jnp.float32),
            pltpu.SemaphoreType.DMA,
            pltpu.VMEM_SHARED((N_HALF, D), jnp.float32),
        ],
    )()
    return f(src, dst, posx, posy, posz, phi, tab, zer)


def _spg_body(qn_hbm, idx_hbm, out_hbm, idx_v, rows_v, sem):
    cid = lax.axis_index("c")
    sid = lax.axis_index("s")
    wid = sid * NC + cid

    @pl.when(wid < NSP // 8)
    def _():
        base = wid * 8
        pltpu.sync_copy(idx_hbm.at[pl.ds(base, 8)], idx_v)
        pltpu.async_copy(qn_hbm.at[idx_v], rows_v, sem).wait()
        pltpu.sync_copy(rows_v, out_hbm.at[pl.ds(base, 8)])


def _sc_spgather(qn, spidx):
    mesh = plsc.VectorSubcoreMesh(core_axis_name="c", subcore_axis_name="s")
    f = functools.partial(
        pl.kernel, _spg_body,
        out_type=jax.ShapeDtypeStruct((NSP, D), jnp.float32),
        mesh=mesh,
        compiler_params=pltpu.CompilerParams(needs_layout_passes=False),
        scratch_types=[
            pltpu.VMEM((8,), jnp.int32),
            pltpu.VMEM((8, D), jnp.float32),
            pltpu.SemaphoreType.DMA,
        ],
    )()
    return f(qn, spidx)


def kernel(pos, atomic_numbers, edge_index, spnode_idx, emb, W1, b1, W2, b2,
           Wf, bf, Wproj):
    f32 = jnp.float32
    src = edge_index[0].astype(jnp.int32)
    dst = edge_index[1].astype(jnp.int32)
    posx = pos[:, 0].astype(f32)
    posy = pos[:, 1].astype(f32)
    posz = pos[:, 2].astype(f32)
    emb_p = jnp.zeros((128, D), f32).at[:emb.shape[0]].set(emb)
    W2a = W2[:, :D]
    b2a = b2[:D].reshape(1, D)
    wfa_pad = jnp.concatenate(
        [Wf[:, :D], bf[None, :D], jnp.zeros((32 - NRBF - 1, D), f32)], axis=0)
    zer = jnp.zeros((ROWS_PT, D), f32)
    an2 = atomic_numbers.astype(jnp.int32).reshape(N, 1)

    q, phi = _dense1(an2, emb_p, W1, b1.reshape(1, D), W2a, b2a)
    tab = _build_table(wfa_pad)
    parts = _sc_edges(src, dst, posx, posy, posz, phi, tab, zer)
    atom_feat, qn = _dense2(q, parts, Wproj)
    spnode_feat = _sc_spgather(qn, spnode_idx.astype(jnp.int32))
    return atom_feat, spnode_feat


# speedup vs baseline: 1.7384x; 1.1924x over previous
import functools

import jax
import jax.numpy as jnp
from jax import lax
from jax.experimental import pallas as pl
from jax.experimental.pallas import tpu as pltpu
from jax.experimental.pallas import tpu_sc as plsc

N = 10000
E = 320000
D = 128
NRBF = 20
CUTOFF = 12.0
NSP = 64

NC = 2
NS = 16
L = 16
NW = NC * NS
EPT = E // NS
CH1 = 2000
B2 = 32
SURV_CAP = EPT + 128
N_PAD = 10240
N_HALF = N_PAD // NC
ROWS_PT = N_HALF // NS

_SIGMA = CUTOFF / NRBF
_NEG_INV_2S2 = -1.0 / (2.0 * _SIGMA * _SIGMA)
_DELTA = CUTOFF / (NRBF - 1)
_PI = 3.14159265358979323846
_CUT2 = CUTOFF * CUTOFF
_DPACK = 8192
_PADV = 1 << 30
G = 128
_GSCALE = (G - 1) / CUTOFF

_RB = 1000


def _d1_body(an_ref, emb_ref, w1_ref, b1_ref, w2_ref, b2_ref, q_ref, phi_ref):
    an = an_ref[...]
    io = lax.broadcasted_iota(jnp.int32, (_RB, 128), 1)
    oh = (io == an).astype(jnp.float32)
    q = jnp.dot(oh, emb_ref[...], preferred_element_type=jnp.float32)
    h = jnp.dot(q, w1_ref[...], preferred_element_type=jnp.float32) + b1_ref[...]
    h = h * jax.nn.sigmoid(h)
    phi_ref[...] = (jnp.dot(h, w2_ref[...], preferred_element_type=jnp.float32)
                    + b2_ref[...])
    q_ref[...] = q


def _dense1(an2, emb_p, W1, b1r, W2a, b2r):
    return pl.pallas_call(
        _d1_body,
        grid=(N // _RB,),
        in_specs=[
            pl.BlockSpec((_RB, 1), lambda i: (i, 0)),
            pl.BlockSpec((128, D), lambda i: (0, 0)),
            pl.BlockSpec((D, D), lambda i: (0, 0)),
            pl.BlockSpec((1, D), lambda i: (0, 0)),
            pl.BlockSpec((D, D), lambda i: (0, 0)),
            pl.BlockSpec((1, D), lambda i: (0, 0)),
        ],
        out_specs=[
            pl.BlockSpec((_RB, D), lambda i: (i, 0)),
            pl.BlockSpec((_RB, D), lambda i: (i, 0)),
        ],
        out_shape=[
            jax.ShapeDtypeStruct((N, D), jnp.float32),
            jax.ShapeDtypeStruct((N, D), jnp.float32),
        ],
    )(an2, emb_p, W1, b1r, W2a, b2r)


def _d2_body(q_ref, parts_ref, wproj_ref, af_ref, qn_ref):
    qn = q_ref[...] + parts_ref[...]
    af_ref[...] = jnp.dot(qn, wproj_ref[...], preferred_element_type=jnp.float32)
    qn_ref[...] = qn


def _dense2(q, parts, Wproj):
    return pl.pallas_call(
        _d2_body,
        grid=(N // _RB,),
        in_specs=[
            pl.BlockSpec((_RB, D), lambda i: (i, 0)),
            pl.BlockSpec((_RB, D), lambda i: (i, 0)),
            pl.BlockSpec((D, D), lambda i: (0, 0)),
        ],
        out_specs=[
            pl.BlockSpec((_RB, D), lambda i: (i, 0)),
            pl.BlockSpec((_RB, D), lambda i: (i, 0)),
        ],
        out_shape=[
            jax.ShapeDtypeStruct((N, D), jnp.float32),
            jax.ShapeDtypeStruct((N, D), jnp.float32),
        ],
    )(q, parts, Wproj)


def _tab_body(wfa_ref, tab_ref):
    dg = (lax.broadcasted_iota(jnp.int32, (G, 32), 0).astype(jnp.float32)
          * (CUTOFF / (G - 1)))
    ci = lax.broadcasted_iota(jnp.int32, (G, 32), 1)
    cif = ci.astype(jnp.float32)
    rbf = jnp.exp(-((dg - cif * _DELTA) ** 2) * (-_NEG_INV_2S2))
    basis = jnp.where(ci < NRBF, rbf,
                      jnp.where(ci == NRBF, 1.0, 0.0))
    dgc = dg[:, :1]
    fc = 0.5 * (jnp.cos(dgc * (_PI / CUTOFF)) + 1.0)
    fc = fc * (dgc < CUTOFF).astype(jnp.float32)
    t = jnp.dot(basis, wfa_ref[...], preferred_element_type=jnp.float32) * fc
    rows = lax.broadcasted_iota(jnp.int32, (G, D), 0)
    tnext = jnp.concatenate([lax.slice(t, (1, 0), (G, D)),
                             jnp.zeros((1, D), jnp.float32)], axis=0)
    delta = (tnext - t) * (rows < G - 1).astype(jnp.float32)
    hi = lax.bitcast_convert_type(t.astype(jnp.bfloat16),
                                  jnp.uint16).astype(jnp.int32)
    lo = lax.bitcast_convert_type(delta.astype(jnp.bfloat16),
                                  jnp.uint16).astype(jnp.int32)
    tab_ref[...] = (hi << 16) | lo


def _build_table(wfa_pad):
    return pl.pallas_call(
        _tab_body,
        grid=(1,),
        in_specs=[pl.BlockSpec((32, D), lambda i: (0, 0))],
        out_specs=pl.BlockSpec((G, D), lambda i: (0, 0)),
        out_shape=jax.ShapeDtypeStruct((G, D), jnp.int32),
    )(wfa_pad)


def _edge_body(src_hbm, dst_hbm, posx_hbm, posy_hbm, posz_hbm, phi_hbm,
               tab_hbm, zer_hbm, out_hbm,
               posx_v, posy_v, posz_v, tab_v, srcbuf, dstbuf, spack,
               srcidx0, idx2d0, phibuf0, outbuf0,
               srcidx1, idx2d1, phibuf1, outbuf1,
               semg0, sems0, semg1, sems1, acc):
    cid = lax.axis_index("c")
    sid = lax.axis_index("s")
    lo = cid * N_HALF

    pltpu.sync_copy(posx_hbm, posx_v)
    pltpu.sync_copy(posy_hbm, posy_v)
    pltpu.sync_copy(posz_hbm, posz_v)
    pltpu.sync_copy(tab_hbm, tab_v)
    coffs = [lax.iota(jnp.int32, L) + c * L for c in range(D // L)]
    pltpu.sync_copy(zer_hbm, acc.at[pl.ds(sid * ROWS_PT, ROWS_PT)])

    ebase = sid * EPT

    def chunk1(c, cnt):
        pltpu.sync_copy(src_hbm.at[pl.ds(ebase + c * CH1, CH1)], srcbuf)
        pltpu.sync_copy(dst_hbm.at[pl.ds(ebase + c * CH1, CH1)], dstbuf)

        def vreg5(i5, cnt):
            for u in range(5):
                cnt = scan16(i5 * 5 + u, cnt)
            return cnt

        def scan16(i, cnt):
            s16 = srcbuf[pl.ds(i * L, L)]
            d16 = dstbuf[pl.ds(i * L, L)]
            xs = plsc.load_gather(posx_v, [s16])
            xd = plsc.load_gather(posx_v, [d16])
            ys = plsc.load_gather(posy_v, [s16])
            yd = plsc.load_gather(posy_v, [d16])
            zs = plsc.load_gather(posz_v, [s16])
            zd = plsc.load_gather(posz_v, [d16])
            dx = xd - xs
            dy = yd - ys
            dz = zd - zs
            d2 = dx * dx + dy * dy + dz * dz + 1e-12
            d16r = d16 - lo
            m = ((d2 < _CUT2) & (d16r >= 0)) & (d16r < N_HALF)
            pk = (s16 << 13) | d16r
            plsc.store_compressed(spack.at[pl.ds(cnt, L)], pk, mask=m)
            cntv = plsc.all_reduce_population_count(m)
            return cnt + cntv[0]

        return lax.fori_loop(0, CH1 // L // 5, vreg5, cnt)

    cnt = lax.fori_loop(0, EPT // CH1, chunk1, jnp.int32(0))

    padv = jnp.full((L,), _PADV, jnp.int32)
    for j in range(2):
        spack[pl.ds(cnt + j * L, L)] = padv

    plsc.subcore_barrier()

    nch = (cnt + B2 - 1) // B2
    bufs = ((srcidx0, idx2d0, phibuf0, outbuf0, semg0, sems0),
            (srcidx1, idx2d1, phibuf1, outbuf1, semg1, sems1))

    def fire_gather(ch, p):
        srcidx, idx2d, phibuf, _, semg, _ = bufs[p]
        base = ch * B2
        for j in range(B2 // L):
            pk = spack[pl.ds(base + j * L, L)]
            m0 = pk < _PADV
            srcidx[pl.ds(j * L, L)] = jnp.where(m0, pk >> 13, 0)
            idx2d[0, pl.ds(j * L, L)] = jnp.where(m0, pk & (_DPACK - 1), 0)
        pltpu.async_copy(phi_hbm.at[srcidx], phibuf, semg)

    def process(ch, p):
        srcidx, idx2d, phibuf, outbuf, semg, _ = bufs[p]
        _, oidx2d, _, ooutbuf, _, osems = bufs[1 - p]
        pltpu.make_async_copy(phi_hbm.at[srcidx], phibuf, semg).wait()
        @pl.when(ch >= 1)
        def _():
            pltpu.make_async_copy(ooutbuf, acc.at[oidx2d.at[0]], osems).wait()

        @pl.when(ch + 1 < nch)
        def _():
            fire_gather(ch + 1, 1 - p)

        base = ch * B2

        def group(g, _):
            pk = spack[pl.ds(base + g * L, L)]
            m = pk < _PADV
            s16 = srcidx[pl.ds(g * L, L)]
            d16 = idx2d[0, pl.ds(g * L, L)] + lo
            xs = plsc.load_gather(posx_v, [s16])
            xd = plsc.load_gather(posx_v, [d16])
            ys = plsc.load_gather(posy_v, [s16])
            yd = plsc.load_gather(posy_v, [d16])
            zs = plsc.load_gather(posz_v, [s16])
            zd = plsc.load_gather(posz_v, [d16])
            dx = xd - xs
            dy = yd - ys
            dz = zd - zs
            d2v = dx * dx + dy * dy + dz * dz + 1e-12
            ii = plsc.bitcast(d2v, jnp.int32)
            y = plsc.bitcast(jnp.int32(0x5F3759DF) - (ii >> 1), jnp.float32)
            for _i in range(3):
                y = y * (1.5 - 0.5 * d2v * y * y)
            dist = d2v * y
            u = dist * _GSCALE
            iv = u.astype(jnp.int32)
            isel = jnp.where(m, iv, G - 1)
            frac = u - isel.astype(jnp.float32)

            for j in range(L):
                row = g * L + j
                sel = jnp.zeros((L,), jnp.int32) + j
                ib = isel.at[sel].get(mode="promise_in_bounds")
                fb = frac.at[sel].get(mode="promise_in_bounds")
                ws = [plsc.load_gather(tab_v, [ib, coffs[c]])
                      for c in range(D // L)]
                phs = [phibuf[row, pl.ds(c * L, L)] for c in range(D // L)]
                for c in range(D // L):
                    val = plsc.bitcast(ws[c] & jnp.int32(-65536), jnp.float32)
                    dlt = plsc.bitcast(ws[c] << 16, jnp.float32)
                    outbuf[row, pl.ds(c * L, L)] = phs[c] * (val + fb * dlt)
            return 0

        lax.fori_loop(0, B2 // L, group, 0)
        pltpu.async_copy(outbuf, acc.at[idx2d.at[0]], bufs[p][5], add=True)

    @pl.when(nch > 0)
    def _():
        fire_gather(0, 0)

    def pair(pp, _):
        ch0 = pp * 2
        process(ch0, 0)

        @pl.when(ch0 + 1 < nch)
        def _():
            process(ch0 + 1, 1)
        return 0

    lax.fori_loop(0, (nch + 1) // 2, pair, 0)

    @pl.when(nch > 0)
    def _():
        lp = (nch - 1) % 2

        @pl.when(lp == 0)
        def _():
            pltpu.make_async_copy(outbuf0, acc.at[idx2d0.at[0]],
                                  sems0).wait()

        @pl.when(lp == 1)
        def _():
            pltpu.make_async_copy(outbuf1, acc.at[idx2d1.at[0]],
                                  sems1).wait()

    plsc.subcore_barrier()
    pltpu.sync_copy(acc.at[pl.ds(sid * ROWS_PT, ROWS_PT)],
                    out_hbm.at[pl.ds(lo + sid * ROWS_PT, ROWS_PT)])


def _sc_edges(src, dst, posx, posy, posz, phi, tab, zer):
    mesh = plsc.VectorSubcoreMesh(core_axis_name="c", subcore_axis_name="s")
    f = functools.partial(
        pl.kernel, _edge_body,
        out_type=jax.ShapeDtypeStruct((N_PAD, D), jnp.float32),
        mesh=mesh,
        compiler_params=pltpu.CompilerParams(needs_layout_passes=False),
        scratch_types=[
            pltpu.VMEM((N,), jnp.float32),
            pltpu.VMEM((N,), jnp.float32),
            pltpu.VMEM((N,), jnp.float32),
            pltpu.VMEM((G, D), jnp.int32),
            pltpu.VMEM((CH1,), jnp.int32),
            pltpu.VMEM((CH1,), jnp.int32),
            pltpu.VMEM((SURV_CAP,), jnp.int32),
            pltpu.VMEM((B2,), jnp.int32),
            pltpu.VMEM((1, B2), jnp.int32),
            pltpu.VMEM((B2, D), jnp.float32),
            pltpu.VMEM((B2, D), jnp.float32),
            pltpu.VMEM((B2,), jnp.int32),
            pltpu.VMEM((1, B2), jnp.int32),
            pltpu.VMEM((B2, D), jnp.float32),
            pltpu.VMEM((B2, D), jnp.float32),
            pltpu.SemaphoreType.DMA,
            pltpu.SemaphoreType.DMA,
            pltpu.SemaphoreType.DMA,
            pltpu.SemaphoreType.DMA,
            pltpu.VMEM_SHARED((N_HALF, D), jnp.float32),
        ],
    )()
    return f(src, dst, posx, posy, posz, phi, tab, zer)


def _spg_body(qn_hbm, idx_hbm, out_hbm, idx_v, rows_v, sem):
    cid = lax.axis_index("c")
    sid = lax.axis_index("s")
    wid = sid * NC + cid

    @pl.when(wid < NSP // 8)
    def _():
        base = wid * 8
        pltpu.sync_copy(idx_hbm.at[pl.ds(base, 8)], idx_v)
        pltpu.async_copy(qn_hbm.at[idx_v], rows_v, sem).wait()
        pltpu.sync_copy(rows_v, out_hbm.at[pl.ds(base, 8)])


def _sc_spgather(qn, spidx):
    mesh = plsc.VectorSubcoreMesh(core_axis_name="c", subcore_axis_name="s")
    f = functools.partial(
        pl.kernel, _spg_body,
        out_type=jax.ShapeDtypeStruct((NSP, D), jnp.float32),
        mesh=mesh,
        compiler_params=pltpu.CompilerParams(needs_layout_passes=False),
        scratch_types=[
            pltpu.VMEM((8,), jnp.int32),
            pltpu.VMEM((8, D), jnp.float32),
            pltpu.SemaphoreType.DMA,
        ],
    )()
    return f(qn, spidx)


def kernel(pos, atomic_numbers, edge_index, spnode_idx, emb, W1, b1, W2, b2,
           Wf, bf, Wproj):
    f32 = jnp.float32
    src = edge_index[0].astype(jnp.int32)
    dst = edge_index[1].astype(jnp.int32)
    posx = pos[:, 0].astype(f32)
    posy = pos[:, 1].astype(f32)
    posz = pos[:, 2].astype(f32)
    emb_p = jnp.zeros((128, D), f32).at[:emb.shape[0]].set(emb)
    W2a = W2[:, :D]
    b2a = b2[:D].reshape(1, D)
    wfa_pad = jnp.concatenate(
        [Wf[:, :D], bf[None, :D], jnp.zeros((32 - NRBF - 1, D), f32)], axis=0)
    zer = jnp.zeros((ROWS_PT, D), f32)
    an2 = atomic_numbers.astype(jnp.int32).reshape(N, 1)

    q, phi = _dense1(an2, emb_p, W1, b1.reshape(1, D), W2a, b2a)
    tab = _build_table(wfa_pad)
    parts = _sc_edges(src, dst, posx, posy, posz, phi, tab, zer)
    atom_feat, qn = _dense2(q, parts, Wproj)
    spnode_feat = _sc_spgather(qn, spnode_idx.astype(jnp.int32))
    return atom_feat, spnode_feat


# speedup vs baseline: 1.8593x; 1.0695x over previous
import functools

import jax
import jax.numpy as jnp
from jax import lax
from jax.experimental import pallas as pl
from jax.experimental.pallas import tpu as pltpu
from jax.experimental.pallas import tpu_sc as plsc

N = 10000
E = 320000
D = 128
NRBF = 20
CUTOFF = 12.0
NSP = 64

NC = 2
NS = 16
L = 16
NW = NC * NS
EPT = E // NS
CH1 = 1000
B2 = 32
SURV_CAP = EPT + 128
N_PAD = 10240
N_HALF = N_PAD // NC
ROWS_PT = N_HALF // NS

_SIGMA = CUTOFF / NRBF
_NEG_INV_2S2 = -1.0 / (2.0 * _SIGMA * _SIGMA)
_DELTA = CUTOFF / (NRBF - 1)
_PI = 3.14159265358979323846
_CUT2 = CUTOFF * CUTOFF
_DPACK = 8192
_PADV = 1 << 30
G = 128
_GSCALE = (G - 1) / CUTOFF

_RB = 1000


def _d1_body(an_ref, emb_ref, w1_ref, b1_ref, w2_ref, b2_ref, q_ref, phi_ref):
    an = an_ref[...]
    io = lax.broadcasted_iota(jnp.int32, (_RB, 128), 1)
    oh = (io == an).astype(jnp.float32)
    q = jnp.dot(oh, emb_ref[...], preferred_element_type=jnp.float32)
    h = jnp.dot(q, w1_ref[...], preferred_element_type=jnp.float32) + b1_ref[...]
    h = h * jax.nn.sigmoid(h)
    phi_ref[...] = (jnp.dot(h, w2_ref[...], preferred_element_type=jnp.float32)
                    + b2_ref[...])
    q_ref[...] = q


def _dense1(an2, emb_p, W1, b1r, W2a, b2r):
    return pl.pallas_call(
        _d1_body,
        grid=(N // _RB,),
        in_specs=[
            pl.BlockSpec((_RB, 1), lambda i: (i, 0)),
            pl.BlockSpec((128, D), lambda i: (0, 0)),
            pl.BlockSpec((D, D), lambda i: (0, 0)),
            pl.BlockSpec((1, D), lambda i: (0, 0)),
            pl.BlockSpec((D, D), lambda i: (0, 0)),
            pl.BlockSpec((1, D), lambda i: (0, 0)),
        ],
        out_specs=[
            pl.BlockSpec((_RB, D), lambda i: (i, 0)),
            pl.BlockSpec((_RB, D), lambda i: (i, 0)),
        ],
        out_shape=[
            jax.ShapeDtypeStruct((N, D), jnp.float32),
            jax.ShapeDtypeStruct((N, D), jnp.float32),
        ],
    )(an2, emb_p, W1, b1r, W2a, b2r)


def _d2_body(q_ref, parts_ref, wproj_ref, af_ref, qn_ref):
    qn = q_ref[...] + parts_ref[...]
    af_ref[...] = jnp.dot(qn, wproj_ref[...], preferred_element_type=jnp.float32)
    qn_ref[...] = qn


def _dense2(q, parts, Wproj):
    return pl.pallas_call(
        _d2_body,
        grid=(N // _RB,),
        in_specs=[
            pl.BlockSpec((_RB, D), lambda i: (i, 0)),
            pl.BlockSpec((_RB, D), lambda i: (i, 0)),
            pl.BlockSpec((D, D), lambda i: (0, 0)),
        ],
        out_specs=[
            pl.BlockSpec((_RB, D), lambda i: (i, 0)),
            pl.BlockSpec((_RB, D), lambda i: (i, 0)),
        ],
        out_shape=[
            jax.ShapeDtypeStruct((N, D), jnp.float32),
            jax.ShapeDtypeStruct((N, D), jnp.float32),
        ],
    )(q, parts, Wproj)


def _tab_body(wfa_ref, tab_ref):
    dg = (lax.broadcasted_iota(jnp.int32, (G, 32), 0).astype(jnp.float32)
          * (CUTOFF / (G - 1)))
    ci = lax.broadcasted_iota(jnp.int32, (G, 32), 1)
    cif = ci.astype(jnp.float32)
    rbf = jnp.exp(-((dg - cif * _DELTA) ** 2) * (-_NEG_INV_2S2))
    basis = jnp.where(ci < NRBF, rbf,
                      jnp.where(ci == NRBF, 1.0, 0.0))
    dgc = dg[:, :1]
    fc = 0.5 * (jnp.cos(dgc * (_PI / CUTOFF)) + 1.0)
    fc = fc * (dgc < CUTOFF).astype(jnp.float32)
    t = jnp.dot(basis, wfa_ref[...], preferred_element_type=jnp.float32) * fc
    rows = lax.broadcasted_iota(jnp.int32, (G, D), 0)
    tnext = jnp.concatenate([lax.slice(t, (1, 0), (G, D)),
                             jnp.zeros((1, D), jnp.float32)], axis=0)
    delta = (tnext - t) * (rows < G - 1).astype(jnp.float32)
    hi = lax.bitcast_convert_type(t.astype(jnp.bfloat16),
                                  jnp.uint16).astype(jnp.int32)
    lo = lax.bitcast_convert_type(delta.astype(jnp.bfloat16),
                                  jnp.uint16).astype(jnp.int32)
    tab_ref[...] = (hi << 16) | lo


def _build_table(wfa_pad):
    return pl.pallas_call(
        _tab_body,
        grid=(1,),
        in_specs=[pl.BlockSpec((32, D), lambda i: (0, 0))],
        out_specs=pl.BlockSpec((G, D), lambda i: (0, 0)),
        out_shape=jax.ShapeDtypeStruct((G, D), jnp.int32),
    )(wfa_pad)


def _edge_body(src_hbm, dst_hbm, posx_hbm, posy_hbm, posz_hbm, phi_hbm,
               tab_hbm, zer_hbm, out_hbm,
               posx_v, posy_v, posz_v, tab_v,
               srcbuf0, dstbuf0, srcbuf1, dstbuf1, spack,
               srcidx0, idx2d0, phibuf0, outbuf0,
               srcidx1, idx2d1, phibuf1, outbuf1,
               semg0, sems0, semg1, sems1, seme0, seme1, acc):
    cid = lax.axis_index("c")
    sid = lax.axis_index("s")
    lo = cid * N_HALF

    pltpu.sync_copy(posx_hbm, posx_v)
    pltpu.sync_copy(posy_hbm, posy_v)
    pltpu.sync_copy(posz_hbm, posz_v)
    pltpu.sync_copy(tab_hbm, tab_v)
    coffs = [lax.iota(jnp.int32, L) + c * L for c in range(D // L)]
    pltpu.sync_copy(zer_hbm, acc.at[pl.ds(sid * ROWS_PT, ROWS_PT)])

    ebase = sid * EPT

    ebufs = ((srcbuf0, dstbuf0, seme0), (srcbuf1, dstbuf1, seme1))
    NCH1 = EPT // CH1

    def fire_edges(c, p):
        sb, db, sme = ebufs[p]
        pltpu.async_copy(src_hbm.at[pl.ds(ebase + c * CH1, CH1)], sb, sme)
        pltpu.async_copy(dst_hbm.at[pl.ds(ebase + c * CH1, CH1)], db, sme)

    def scan_chunk(c, p, cnt):
        srcbuf, dstbuf, sme = ebufs[p]
        pltpu.make_async_copy(src_hbm.at[pl.ds(ebase, CH1)], srcbuf,
                              sme).wait()
        pltpu.make_async_copy(dst_hbm.at[pl.ds(ebase, CH1)], dstbuf,
                              sme).wait()

        def vreg5(i5, cnt):
            for u in range(5):
                cnt = scan16(i5 * 5 + u, cnt)
            return cnt

        def scan16(i, cnt):
            s16 = srcbuf[pl.ds(i * L, L)]
            d16 = dstbuf[pl.ds(i * L, L)]
            xs = plsc.load_gather(posx_v, [s16])
            xd = plsc.load_gather(posx_v, [d16])
            ys = plsc.load_gather(posy_v, [s16])
            yd = plsc.load_gather(posy_v, [d16])
            zs = plsc.load_gather(posz_v, [s16])
            zd = plsc.load_gather(posz_v, [d16])
            dx = xd - xs
            dy = yd - ys
            dz = zd - zs
            d2 = dx * dx + dy * dy + dz * dz + 1e-12
            d16r = d16 - lo
            m = ((d2 < _CUT2) & (d16r >= 0)) & (d16r < N_HALF)
            pk = (s16 << 13) | d16r
            plsc.store_compressed(spack.at[pl.ds(cnt, L)], pk, mask=m)
            cntv = plsc.all_reduce_population_count(m)
            return cnt + cntv[0]

        cnt = lax.fori_loop(0, CH1 // L // 5, vreg5, cnt)

        @pl.when(c + 2 < NCH1)
        def _():
            fire_edges(c + 2, p)

        return cnt

    fire_edges(0, 0)
    fire_edges(1, 1)

    def pair1(pp, cnt):
        c0 = pp * 2
        cnt = scan_chunk(c0, 0, cnt)
        cnt = scan_chunk(c0 + 1, 1, cnt)
        return cnt

    cnt = lax.fori_loop(0, NCH1 // 2, pair1, jnp.int32(0))

    padv = jnp.full((L,), _PADV, jnp.int32)
    for j in range(2):
        spack[pl.ds(cnt + j * L, L)] = padv

    plsc.subcore_barrier()

    nch = (cnt + B2 - 1) // B2
    bufs = ((srcidx0, idx2d0, phibuf0, outbuf0, semg0, sems0),
            (srcidx1, idx2d1, phibuf1, outbuf1, semg1, sems1))

    def fire_gather(ch, p):
        srcidx, idx2d, phibuf, _, semg, _ = bufs[p]
        base = ch * B2
        for j in range(B2 // L):
            pk = spack[pl.ds(base + j * L, L)]
            m0 = pk < _PADV
            srcidx[pl.ds(j * L, L)] = jnp.where(m0, pk >> 13, 0)
            idx2d[0, pl.ds(j * L, L)] = jnp.where(m0, pk & (_DPACK - 1), 0)
        pltpu.async_copy(phi_hbm.at[srcidx], phibuf, semg)

    def process(ch, p):
        srcidx, idx2d, phibuf, outbuf, semg, _ = bufs[p]
        _, oidx2d, _, ooutbuf, _, osems = bufs[1 - p]
        pltpu.make_async_copy(phi_hbm.at[srcidx], phibuf, semg).wait()
        @pl.when(ch >= 1)
        def _():
            pltpu.make_async_copy(ooutbuf, acc.at[oidx2d.at[0]], osems).wait()

        @pl.when(ch + 1 < nch)
        def _():
            fire_gather(ch + 1, 1 - p)

        base = ch * B2

        def group(g, _):
            pk = spack[pl.ds(base + g * L, L)]
            m = pk < _PADV
            s16 = srcidx[pl.ds(g * L, L)]
            d16 = idx2d[0, pl.ds(g * L, L)] + lo
            xs = plsc.load_gather(posx_v, [s16])
            xd = plsc.load_gather(posx_v, [d16])
            ys = plsc.load_gather(posy_v, [s16])
            yd = plsc.load_gather(posy_v, [d16])
            zs = plsc.load_gather(posz_v, [s16])
            zd = plsc.load_gather(posz_v, [d16])
            dx = xd - xs
            dy = yd - ys
            dz = zd - zs
            d2v = dx * dx + dy * dy + dz * dz + 1e-12
            ii = plsc.bitcast(d2v, jnp.int32)
            y = plsc.bitcast(jnp.int32(0x5F3759DF) - (ii >> 1), jnp.float32)
            for _i in range(3):
                y = y * (1.5 - 0.5 * d2v * y * y)
            dist = d2v * y
            u = dist * _GSCALE
            iv = u.astype(jnp.int32)
            isel = jnp.where(m, iv, G - 1)
            frac = u - isel.astype(jnp.float32)

            for j in range(L):
                row = g * L + j
                sel = jnp.zeros((L,), jnp.int32) + j
                ib = isel.at[sel].get(mode="promise_in_bounds")
                fb = frac.at[sel].get(mode="promise_in_bounds")
                ws = [plsc.load_gather(tab_v, [ib, coffs[c]])
                      for c in range(D // L)]
                phs = [phibuf[row, pl.ds(c * L, L)] for c in range(D // L)]
                for c in range(D // L):
                    val = plsc.bitcast(ws[c] & jnp.int32(-65536), jnp.float32)
                    dlt = plsc.bitcast(ws[c] << 16, jnp.float32)
                    outbuf[row, pl.ds(c * L, L)] = phs[c] * (val + fb * dlt)
            return 0

        lax.fori_loop(0, B2 // L, group, 0)
        pltpu.async_copy(outbuf, acc.at[idx2d.at[0]], bufs[p][5], add=True)

    @pl.when(nch > 0)
    def _():
        fire_gather(0, 0)

    def pair(pp, _):
        ch0 = pp * 2
        process(ch0, 0)

        @pl.when(ch0 + 1 < nch)
        def _():
            process(ch0 + 1, 1)
        return 0

    lax.fori_loop(0, (nch + 1) // 2, pair, 0)

    @pl.when(nch > 0)
    def _():
        lp = (nch - 1) % 2

        @pl.when(lp == 0)
        def _():
            pltpu.make_async_copy(outbuf0, acc.at[idx2d0.at[0]],
                                  sems0).wait()

        @pl.when(lp == 1)
        def _():
            pltpu.make_async_copy(outbuf1, acc.at[idx2d1.at[0]],
                                  sems1).wait()

    plsc.subcore_barrier()
    pltpu.sync_copy(acc.at[pl.ds(sid * ROWS_PT, ROWS_PT)],
                    out_hbm.at[pl.ds(lo + sid * ROWS_PT, ROWS_PT)])


def _sc_edges(src, dst, posx, posy, posz, phi, tab, zer):
    mesh = plsc.VectorSubcoreMesh(core_axis_name="c", subcore_axis_name="s")
    f = functools.partial(
        pl.kernel, _edge_body,
        out_type=jax.ShapeDtypeStruct((N_PAD, D), jnp.float32),
        mesh=mesh,
        compiler_params=pltpu.CompilerParams(needs_layout_passes=False),
        scratch_types=[
            pltpu.VMEM((N,), jnp.float32),
            pltpu.VMEM((N,), jnp.float32),
            pltpu.VMEM((N,), jnp.float32),
            pltpu.VMEM((G, D), jnp.int32),
            pltpu.VMEM((CH1,), jnp.int32),
            pltpu.VMEM((CH1,), jnp.int32),
            pltpu.VMEM((CH1,), jnp.int32),
            pltpu.VMEM((CH1,), jnp.int32),
            pltpu.VMEM((SURV_CAP,), jnp.int32),
            pltpu.VMEM((B2,), jnp.int32),
            pltpu.VMEM((1, B2), jnp.int32),
            pltpu.VMEM((B2, D), jnp.float32),
            pltpu.VMEM((B2, D), jnp.float32),
            pltpu.VMEM((B2,), jnp.int32),
            pltpu.VMEM((1, B2), jnp.int32),
            pltpu.VMEM((B2, D), jnp.float32),
            pltpu.VMEM((B2, D), jnp.float32),
            pltpu.SemaphoreType.DMA,
            pltpu.SemaphoreType.DMA,
            pltpu.SemaphoreType.DMA,
            pltpu.SemaphoreType.DMA,
            pltpu.SemaphoreType.DMA,
            pltpu.SemaphoreType.DMA,
            pltpu.VMEM_SHARED((N_HALF, D), jnp.float32),
        ],
    )()
    return f(src, dst, posx, posy, posz, phi, tab, zer)


def _spg_body(qn_hbm, idx_hbm, out_hbm, idx_v, rows_v, sem):
    cid = lax.axis_index("c")
    sid = lax.axis_index("s")
    wid = sid * NC + cid

    @pl.when(wid < NSP // 8)
    def _():
        base = wid * 8
        pltpu.sync_copy(idx_hbm.at[pl.ds(base, 8)], idx_v)
        pltpu.async_copy(qn_hbm.at[idx_v], rows_v, sem).wait()
        pltpu.sync_copy(rows_v, out_hbm.at[pl.ds(base, 8)])


def _sc_spgather(qn, spidx):
    mesh = plsc.VectorSubcoreMesh(core_axis_name="c", subcore_axis_name="s")
    f = functools.partial(
        pl.kernel, _spg_body,
        out_type=jax.ShapeDtypeStruct((NSP, D), jnp.float32),
        mesh=mesh,
        compiler_params=pltpu.CompilerParams(needs_layout_passes=False),
        scratch_types=[
            pltpu.VMEM((8,), jnp.int32),
            pltpu.VMEM((8, D), jnp.float32),
            pltpu.SemaphoreType.DMA,
        ],
    )()
    return f(qn, spidx)


def kernel(pos, atomic_numbers, edge_index, spnode_idx, emb, W1, b1, W2, b2,
           Wf, bf, Wproj):
    f32 = jnp.float32
    src = edge_index[0].astype(jnp.int32)
    dst = edge_index[1].astype(jnp.int32)
    posx = pos[:, 0].astype(f32)
    posy = pos[:, 1].astype(f32)
    posz = pos[:, 2].astype(f32)
    emb_p = jnp.zeros((128, D), f32).at[:emb.shape[0]].set(emb)
    W2a = W2[:, :D]
    b2a = b2[:D].reshape(1, D)
    wfa_pad = jnp.concatenate(
        [Wf[:, :D], bf[None, :D], jnp.zeros((32 - NRBF - 1, D), f32)], axis=0)
    zer = jnp.zeros((ROWS_PT, D), f32)
    an2 = atomic_numbers.astype(jnp.int32).reshape(N, 1)

    q, phi = _dense1(an2, emb_p, W1, b1.reshape(1, D), W2a, b2a)
    tab = _build_table(wfa_pad)
    parts = _sc_edges(src, dst, posx, posy, posz, phi, tab, zer)
    atom_feat, qn = _dense2(q, parts, Wproj)
    spnode_feat = _sc_spgather(qn, spnode_idx.astype(jnp.int32))
    return atom_feat, spnode_feat
